# Initial kernel scaffold; baseline (speedup 1.0000x reference)
import functools

import jax
import jax.numpy as jnp
from jax import lax
from jax.experimental import pallas as pl
from jax.experimental.pallas import tpu as pltpu
from jax.experimental.pallas import tpu_sc as plsc

VOCAB = 1_000_000
EMBED_DIM = 32
BATCH = 16384
HIST = 20
IN_DIM = HIST * EMBED_DIM
HIDDEN = 256
OUT_DIM = 128

N_ROWS = BATCH * HIST
NUM_CORES = 2
NUM_SUBCORES = 16
NW = NUM_CORES * NUM_SUBCORES
PER_W = N_ROWS // NW
CHUNK = 2048
NCHUNK = PER_W // CHUNK

_sc_mesh = plsc.VectorSubcoreMesh(core_axis_name="c", subcore_axis_name="s")


@functools.partial(
    pl.kernel,
    mesh=_sc_mesh,
    out_type=jax.ShapeDtypeStruct((N_ROWS, EMBED_DIM), jnp.float32),
    scratch_types=[
        pltpu.VMEM((CHUNK,), jnp.int32),
        pltpu.VMEM((CHUNK, EMBED_DIM), jnp.float32),
        pltpu.SemaphoreType.DMA,
    ],
)
def _sc_gather(idx_hbm, table_hbm, out_hbm, idx_v, rows_v, sem):
    wid = lax.axis_index("s") * NUM_CORES + lax.axis_index("c")
    base = wid * PER_W

    def body(i, carry):
        off = base + i * CHUNK
        pltpu.sync_copy(idx_hbm.at[pl.ds(off, CHUNK)], idx_v)
        pltpu.async_copy(table_hbm.at[idx_v], rows_v, sem).wait()
        pltpu.sync_copy(rows_v, out_hbm.at[pl.ds(off, CHUNK)])
        return carry

    lax.fori_loop(0, NCHUNK, body, 0)


BB = 1024


def _mlp_body(x_ref, w1_ref, b1_ref, w2_ref, o_ref):
    z = jnp.dot(x_ref[...], w1_ref[...], preferred_element_type=jnp.float32)
    z = z + b1_ref[...]
    h = 1.0 / (1.0 + jnp.exp(-z))
    o_ref[...] = jnp.dot(h, w2_ref[...], preferred_element_type=jnp.float32)


_mlp = pl.pallas_call(
    _mlp_body,
    grid=(BATCH // BB,),
    in_specs=[
        pl.BlockSpec((BB, IN_DIM), lambda i: (i, 0)),
        pl.BlockSpec((IN_DIM, HIDDEN), lambda i: (0, 0)),
        pl.BlockSpec((1, HIDDEN), lambda i: (0, 0)),
        pl.BlockSpec((HIDDEN, OUT_DIM), lambda i: (0, 0)),
    ],
    out_specs=pl.BlockSpec((BB, OUT_DIM), lambda i: (i, 0)),
    out_shape=jax.ShapeDtypeStruct((BATCH, OUT_DIM), jnp.float32),
)


def kernel(x, word_vectors, W1, b1, W2):
    idx = x.reshape(-1).astype(jnp.int32)
    emb = _sc_gather(idx, word_vectors)
    win = emb.reshape(BATCH, IN_DIM)
    return _mlp(win, W1, b1.reshape(1, HIDDEN), W2)


# speedup vs baseline: 14.1278x; 14.1278x over previous
import functools

import jax
import jax.numpy as jnp
from jax import lax
from jax.experimental import pallas as pl
from jax.experimental.pallas import tpu as pltpu
from jax.experimental.pallas import tpu_sc as plsc

VOCAB = 1_000_000
EMBED_DIM = 32
BATCH = 16384
HIST = 20
IN_DIM = HIST * EMBED_DIM
HIDDEN = 256
OUT_DIM = 128

N_ROWS = BATCH * HIST
NUM_CORES = 2
NUM_SUBCORES = 16
NW = NUM_CORES * NUM_SUBCORES
PER_W = N_ROWS // NW
CHUNK = 2048
NCHUNK = PER_W // CHUNK

_sc_mesh = plsc.VectorSubcoreMesh(core_axis_name="c", subcore_axis_name="s")


@functools.partial(
    pl.kernel,
    mesh=_sc_mesh,
    out_type=jax.ShapeDtypeStruct((N_ROWS, EMBED_DIM), jnp.float32),
    scratch_types=[
        pltpu.VMEM((CHUNK,), jnp.int32),
        pltpu.VMEM((CHUNK, EMBED_DIM), jnp.float32),
        pltpu.SemaphoreType.DMA,
    ],
    compiler_params=pltpu.CompilerParams(use_tc_tiling_on_sc=False),
)
def _sc_gather(idx_hbm, table_hbm, out_hbm, idx_v, rows_v, sem):
    wid = lax.axis_index("s") * NUM_CORES + lax.axis_index("c")
    base = wid * PER_W

    def body(i, carry):
        off = base + i * CHUNK
        pltpu.sync_copy(idx_hbm.at[pl.ds(off, CHUNK)], idx_v)
        pltpu.async_copy(table_hbm.at[idx_v], rows_v, sem).wait()
        pltpu.sync_copy(rows_v, out_hbm.at[pl.ds(off, CHUNK)])
        return carry

    lax.fori_loop(0, NCHUNK, body, 0)


BB = 1024


def _mlp_body(x_ref, w1_ref, b1_ref, w2_ref, o_ref):
    z = jnp.dot(x_ref[...], w1_ref[...], preferred_element_type=jnp.float32)
    z = z + b1_ref[...]
    h = 1.0 / (1.0 + jnp.exp(-z))
    o_ref[...] = jnp.dot(h, w2_ref[...], preferred_element_type=jnp.float32)


_mlp = pl.pallas_call(
    _mlp_body,
    grid=(BATCH // BB,),
    in_specs=[
        pl.BlockSpec((BB, IN_DIM), lambda i: (i, 0)),
        pl.BlockSpec((IN_DIM, HIDDEN), lambda i: (0, 0)),
        pl.BlockSpec((1, HIDDEN), lambda i: (0, 0)),
        pl.BlockSpec((HIDDEN, OUT_DIM), lambda i: (0, 0)),
    ],
    out_specs=pl.BlockSpec((BB, OUT_DIM), lambda i: (i, 0)),
    out_shape=jax.ShapeDtypeStruct((BATCH, OUT_DIM), jnp.float32),
)


def kernel(x, word_vectors, W1, b1, W2):
    idx = x.reshape(-1).astype(jnp.int32)
    emb = _sc_gather(idx, word_vectors)
    win = emb.reshape(BATCH, IN_DIM)
    return _mlp(win, W1, b1.reshape(1, HIDDEN), W2)


# speedup vs baseline: 14.7774x; 1.0460x over previous
import functools

import jax
import jax.numpy as jnp
from jax import lax
from jax.experimental import pallas as pl
from jax.experimental.pallas import tpu as pltpu
from jax.experimental.pallas import tpu_sc as plsc

VOCAB = 1_000_000
EMBED_DIM = 32
BATCH = 16384
HIST = 20
IN_DIM = HIST * EMBED_DIM
HIDDEN = 256
OUT_DIM = 128

N_ROWS = BATCH * HIST
NUM_CORES = 2
NUM_SUBCORES = 16
NW = NUM_CORES * NUM_SUBCORES
PER_W = N_ROWS // NW
CHUNK = 2560
NCHUNK = PER_W // CHUNK

_sc_mesh = plsc.VectorSubcoreMesh(core_axis_name="c", subcore_axis_name="s")


@functools.partial(
    pl.kernel,
    mesh=_sc_mesh,
    out_type=jax.ShapeDtypeStruct((N_ROWS, EMBED_DIM), jnp.float32),
    scratch_types=[
        pltpu.VMEM((CHUNK,), jnp.int32),
        pltpu.VMEM((CHUNK, EMBED_DIM), jnp.float32),
        pltpu.SemaphoreType.DMA,
    ],
    compiler_params=pltpu.CompilerParams(use_tc_tiling_on_sc=False),
)
def _sc_gather(idx_hbm, table_hbm, out_hbm, idx_v, rows_v, sem):
    wid = lax.axis_index("s") * NUM_CORES + lax.axis_index("c")
    base = wid * PER_W

    def body(i, carry):
        off = base + i * CHUNK
        pltpu.sync_copy(idx_hbm.at[pl.ds(off, CHUNK)], idx_v)
        pltpu.async_copy(table_hbm.at[idx_v], rows_v, sem).wait()
        pltpu.sync_copy(rows_v, out_hbm.at[pl.ds(off, CHUNK)])
        return carry

    lax.fori_loop(0, NCHUNK, body, 0)


BB = 1024
TB = BB // 8
NS = IN_DIM // 128


def _mlp_body(x_ref, w1_ref, b1_ref, w2_ref, o_ref):
    acc = jnp.zeros((BB, HIDDEN), dtype=jnp.float32)
    for ct in range(NS):
        xc = x_ref[:, ct, :, :].reshape(BB, 128)
        acc = acc + jnp.dot(xc, w1_ref[ct], preferred_element_type=jnp.float32)
    z = acc + b1_ref[...]
    h = 1.0 / (1.0 + jnp.exp(-z))
    o_ref[...] = jnp.dot(h, w2_ref[...], preferred_element_type=jnp.float32)


_mlp = pl.pallas_call(
    _mlp_body,
    grid=(BATCH // BB,),
    in_specs=[
        pl.BlockSpec((TB, NS, 8, 128), lambda i: (i, 0, 0, 0)),
        pl.BlockSpec((NS, 128, HIDDEN), lambda i: (0, 0, 0)),
        pl.BlockSpec((1, HIDDEN), lambda i: (0, 0)),
        pl.BlockSpec((HIDDEN, OUT_DIM), lambda i: (0, 0)),
    ],
    out_specs=pl.BlockSpec((BB, OUT_DIM), lambda i: (i, 0)),
    out_shape=jax.ShapeDtypeStruct((BATCH, OUT_DIM), jnp.float32),
)


def kernel(x, word_vectors, W1, b1, W2):
    idxp = (x.astype(jnp.int32)
             .reshape(BATCH // 8, 8, NS, 4)
             .transpose(0, 2, 1, 3)
             .reshape(-1))
    emb = _sc_gather(idxp, word_vectors)
    e4d = emb.reshape(BATCH // 8, NS, 8, 128)
    w1v = W1.reshape(NS, 128, HIDDEN)
    return _mlp(e4d, w1v, b1.reshape(1, HIDDEN), W2)
